# baseline (speedup 1.0000x reference)
import jax
import jax.numpy as jnp
from jax import lax
from jax.experimental import pallas as pl
from jax.experimental.pallas import tpu as pltpu
from jax.experimental.pallas import tpu_sc as plsc

NUM_ENTITIES = 1000000
NUM_RELATIONS = 1000
EMBED_DIM = 32
BATCH = 16384

NC = 2
NS = 16
LANES = 16
NW = NC * NS
BPW = BATCH // NW
CHUNK = 128
NCHUNK = BPW // CHUNK

FULL_COLS = NUM_ENTITIES // 128
TAIL = NUM_ENTITIES - FULL_COLS * 128
SEG = (FULL_COLS + 1) * 128
COLS_LO = FULL_COLS // NW
EXTRA = FULL_COLS - COLS_LO * NW
BLK = 30
NROUNDS = (COLS_LO + 1 + BLK - 1) // BLK


def _detile_body(ent_hbm, tail_hbm, flat_hbm, buf, tbuf, sem, wsem):
    wid = lax.axis_index("s") * NC + lax.axis_index("c")
    ncols = jnp.where(wid < EXTRA, COLS_LO + 1, COLS_LO)
    start = wid * COLS_LO + jnp.minimum(wid, EXTRA)

    def fire_reads(r, p):
        off = jnp.minimum(r * BLK, ncols - BLK)
        for jj in range(BLK):
            j = start + off + jj
            pltpu.async_copy(ent_hbm.at[:, pl.ds(j * 128, 128)],
                             buf.at[p, :, pl.ds(jj * 128, 128)], sem)

    def drain_reads():
        for jj in range(BLK):
            pltpu.make_async_copy(ent_hbm.at[:, pl.ds(0, 128)],
                                  buf.at[0, :, pl.ds(0, 128)], sem).wait()

    def fire_writes(r, p):
        off = jnp.minimum(r * BLK, ncols - BLK)
        for d in range(EMBED_DIM):
            pltpu.async_copy(
                buf.at[p, d],
                flat_hbm.at[pl.ds(d * SEG + (start + off) * 128, BLK * 128)],
                wsem)

    def drain_writes():
        for d in range(EMBED_DIM):
            pltpu.make_async_copy(
                buf.at[0, 0],
                flat_hbm.at[pl.ds(0, BLK * 128)], wsem).wait()

    def round_body(r, carry):
        fire_reads(r, 0)
        drain_reads()
        fire_writes(r, 0)
        drain_writes()
        return carry

    lax.fori_loop(0, NROUNDS, round_body, 0)

    @pl.when(wid == NW - 1)
    def _():
        pltpu.async_copy(tail_hbm, tbuf, sem).wait()

        def tail_wr(d, carry2):
            pltpu.async_copy(
                tbuf.at[pl.ds(d * 128, 128)],
                flat_hbm.at[pl.ds(d * SEG + FULL_COLS * 128, 128)],
                wsem).wait()
            return carry2

        lax.fori_loop(0, EMBED_DIM, tail_wr, 0)


def _score_body(heads_hbm, rels_hbm, tails_hbm, ent_hbm, rel_hbm, out_hbm,
                idx_h, idx_t, idx_r, hcols, tcols, rel_v, out_v, sem, rsem):
    wid = lax.axis_index("s") * NC + lax.axis_index("c")
    base = wid * BPW

    for j in range(NCHUNK):
        src = pl.ds(base + j * CHUNK, CHUNK)
        pltpu.sync_copy(heads_hbm.at[src], idx_h.at[j])
        pltpu.sync_copy(tails_hbm.at[src], idx_t.at[j])
    pltpu.sync_copy(rels_hbm.at[pl.ds(base, BPW)], idx_r)

    rel_copy = pltpu.async_copy(rel_hbm, rel_v, rsem)

    for j in range(NCHUNK):
        dst = pl.ds(j * CHUNK, CHUNK)
        for d in range(EMBED_DIM):
            seg = pl.ds(d * SEG, SEG)
            pltpu.async_copy(ent_hbm.at[seg].at[idx_h.at[j]],
                             hcols.at[d, dst], sem)
            pltpu.async_copy(ent_hbm.at[seg].at[idx_t.at[j]],
                             tcols.at[d, dst], sem)

    rel_copy.wait()

    for j in range(NCHUNK):
        for _ in range(2 * EMBED_DIM):
            pltpu.make_async_copy(ent_hbm.at[pl.ds(0, SEG)].at[idx_h.at[0]],
                                  hcols.at[0, pl.ds(0, CHUNK)], sem).wait()

        def group(g, carry):
            sl = pl.ds(j * CHUNK + g * LANES, LANES)
            ridx = idx_r[sl]
            acc = jnp.zeros((LANES,), jnp.float32)
            for d in range(EMBED_DIM):
                rv = plsc.load_gather(rel_v, [jnp.full((LANES,), d, jnp.int32), ridx])
                acc = acc + hcols[d, sl] * rv * tcols[d, sl]
            out_v[sl] = acc
            return carry

        lax.fori_loop(0, CHUNK // LANES, group, 0)

    pltpu.sync_copy(out_v, out_hbm.at[pl.ds(base, BPW)])


@jax.jit
def _run(heads, relations, tails, ent_t, tail_flat, rel_t):
    mesh = plsc.VectorSubcoreMesh(core_axis_name="c", subcore_axis_name="s")

    detile = pl.kernel(
        _detile_body,
        out_type=jax.ShapeDtypeStruct((SEG * EMBED_DIM,), jnp.float32),
        mesh=mesh,
        compiler_params=pltpu.CompilerParams(needs_layout_passes=False),
        scratch_types=[
            pltpu.VMEM((1, EMBED_DIM, BLK * 128), jnp.float32),
            pltpu.VMEM((EMBED_DIM * 128,), jnp.float32),
            pltpu.SemaphoreType.DMA,
            pltpu.SemaphoreType.DMA,
        ],
    )
    ent_flat = detile(ent_t, tail_flat)

    score = pl.kernel(
        _score_body,
        out_type=jax.ShapeDtypeStruct((BATCH,), jnp.float32),
        mesh=mesh,
        compiler_params=pltpu.CompilerParams(
            needs_layout_passes=False, use_tc_tiling_on_sc=False),
        scratch_types=[
            pltpu.VMEM((NCHUNK, CHUNK), jnp.int32),
            pltpu.VMEM((NCHUNK, CHUNK), jnp.int32),
            pltpu.VMEM((BPW,), jnp.int32),
            pltpu.VMEM((EMBED_DIM, BPW), jnp.float32),
            pltpu.VMEM((EMBED_DIM, BPW), jnp.float32),
            pltpu.VMEM((EMBED_DIM, NUM_RELATIONS), jnp.float32),
            pltpu.VMEM((BPW,), jnp.float32),
            pltpu.SemaphoreType.DMA,
            pltpu.SemaphoreType.DMA,
        ],
    )
    return score(heads, relations, tails, ent_flat, rel_t)


def kernel(heads, relations, tails, entity_table, relation_table):
    return _run(
        heads.astype(jnp.int32),
        relations.astype(jnp.int32),
        tails.astype(jnp.int32),
        entity_table.T,
        jnp.pad(entity_table[FULL_COLS * 128:].T, ((0, 0), (0, 128 - TAIL))).reshape(-1),
        relation_table.T,
    )


# speedup vs baseline: 1.0065x; 1.0065x over previous
import jax
import jax.numpy as jnp
from jax import lax
from jax.experimental import pallas as pl
from jax.experimental.pallas import tpu as pltpu
from jax.experimental.pallas import tpu_sc as plsc

NUM_ENTITIES = 1000000
NUM_RELATIONS = 1000
EMBED_DIM = 32
BATCH = 16384

NC = 2
NS = 16
LANES = 16
NW = NC * NS
BPW = BATCH // NW
CHUNK = 128
NCHUNK = BPW // CHUNK

FULL_COLS = NUM_ENTITIES // 128
TAIL = NUM_ENTITIES - FULL_COLS * 128
SEG = (FULL_COLS + 1) * 128
COLS_LO = FULL_COLS // NW
EXTRA = FULL_COLS - COLS_LO * NW
BLK = 24
NROUNDS = (COLS_LO + 1 + BLK - 1) // BLK


def _detile_body(ent_hbm, tail_hbm, flat_hbm, buf, tbuf, sem, wsem):
    wid = lax.axis_index("s") * NC + lax.axis_index("c")
    ncols = jnp.where(wid < EXTRA, COLS_LO + 1, COLS_LO)
    start = wid * COLS_LO + jnp.minimum(wid, EXTRA)

    def fire_reads(r, p):
        off = jnp.minimum(r * BLK, ncols - BLK)
        for jj in range(BLK):
            j = start + off + jj
            pltpu.async_copy(ent_hbm.at[:, pl.ds(j * 128, 128)],
                             buf.at[p, :, pl.ds(jj * 128, 128)], sem)

    def drain_reads():
        for jj in range(BLK):
            pltpu.make_async_copy(ent_hbm.at[:, pl.ds(0, 128)],
                                  buf.at[0, :, pl.ds(0, 128)], sem).wait()

    def fire_writes(r, p):
        off = jnp.minimum(r * BLK, ncols - BLK)
        for d in range(EMBED_DIM):
            pltpu.async_copy(
                buf.at[p, d],
                flat_hbm.at[pl.ds(d * SEG + (start + off) * 128, BLK * 128)],
                wsem)

    def drain_writes():
        for d in range(EMBED_DIM):
            pltpu.make_async_copy(
                buf.at[0, 0],
                flat_hbm.at[pl.ds(0, BLK * 128)], wsem).wait()

    def round_body(r, carry):
        fire_reads(r, 0)
        drain_reads()
        fire_writes(r, 0)
        drain_writes()
        return carry

    lax.fori_loop(0, NROUNDS, round_body, 0)

    @pl.when(wid == NW - 1)
    def _():
        pltpu.async_copy(tail_hbm, tbuf, sem).wait()

        def tail_wr(d, carry2):
            pltpu.async_copy(
                tbuf.at[pl.ds(d * 128, 128)],
                flat_hbm.at[pl.ds(d * SEG + FULL_COLS * 128, 128)],
                wsem).wait()
            return carry2

        lax.fori_loop(0, EMBED_DIM, tail_wr, 0)


def _score_body(heads_hbm, rels_hbm, tails_hbm, ent_hbm, rel_hbm, out_hbm,
                idx_h, idx_t, idx_r, hcols, tcols, rel_v, out_v, sem, rsem):
    wid = lax.axis_index("s") * NC + lax.axis_index("c")
    base = wid * BPW

    for j in range(NCHUNK):
        src = pl.ds(base + j * CHUNK, CHUNK)
        pltpu.sync_copy(heads_hbm.at[src], idx_h.at[j])
        pltpu.sync_copy(tails_hbm.at[src], idx_t.at[j])
    pltpu.sync_copy(rels_hbm.at[pl.ds(base, BPW)], idx_r)

    rel_copy = pltpu.async_copy(rel_hbm, rel_v, rsem)

    copies = []
    for j in range(NCHUNK):
        dst = pl.ds(j * CHUNK, CHUNK)
        for d in range(EMBED_DIM):
            seg = pl.ds(d * SEG, SEG)
            copies.append(
                pltpu.async_copy(ent_hbm.at[seg].at[idx_h.at[j]],
                                 hcols.at[d, dst], sem))
            copies.append(
                pltpu.async_copy(ent_hbm.at[seg].at[idx_t.at[j]],
                                 tcols.at[d, dst], sem))
    for c in copies:
        c.wait()

    rel_copy.wait()

    def group(g, carry):
        sl = pl.ds(g * LANES, LANES)
        ridx = idx_r[sl]
        acc = jnp.zeros((LANES,), jnp.float32)
        for d in range(EMBED_DIM):
            rv = plsc.load_gather(rel_v, [jnp.full((LANES,), d, jnp.int32), ridx])
            acc = acc + hcols[d, sl] * rv * tcols[d, sl]
        out_v[sl] = acc
        return carry

    lax.fori_loop(0, BPW // LANES, group, 0)

    pltpu.sync_copy(out_v, out_hbm.at[pl.ds(base, BPW)])


@jax.jit
def _run(heads, relations, tails, ent_t, tail_flat, rel_t):
    mesh = plsc.VectorSubcoreMesh(core_axis_name="c", subcore_axis_name="s")

    detile = pl.kernel(
        _detile_body,
        out_type=jax.ShapeDtypeStruct((SEG * EMBED_DIM,), jnp.float32),
        mesh=mesh,
        compiler_params=pltpu.CompilerParams(needs_layout_passes=False),
        scratch_types=[
            pltpu.VMEM((1, EMBED_DIM, BLK * 128), jnp.float32),
            pltpu.VMEM((EMBED_DIM * 128,), jnp.float32),
            pltpu.SemaphoreType.DMA,
            pltpu.SemaphoreType.DMA,
        ],
    )
    ent_flat = detile(ent_t, tail_flat)

    score = pl.kernel(
        _score_body,
        out_type=jax.ShapeDtypeStruct((BATCH,), jnp.float32),
        mesh=mesh,
        compiler_params=pltpu.CompilerParams(
            needs_layout_passes=False, use_tc_tiling_on_sc=False),
        scratch_types=[
            pltpu.VMEM((NCHUNK, CHUNK), jnp.int32),
            pltpu.VMEM((NCHUNK, CHUNK), jnp.int32),
            pltpu.VMEM((BPW,), jnp.int32),
            pltpu.VMEM((EMBED_DIM, BPW), jnp.float32),
            pltpu.VMEM((EMBED_DIM, BPW), jnp.float32),
            pltpu.VMEM((EMBED_DIM, NUM_RELATIONS), jnp.float32),
            pltpu.VMEM((BPW,), jnp.float32),
            pltpu.SemaphoreType.DMA,
            pltpu.SemaphoreType.DMA,
        ],
    )
    return score(heads, relations, tails, ent_flat, rel_t)


def kernel(heads, relations, tails, entity_table, relation_table):
    return _run(
        heads.astype(jnp.int32),
        relations.astype(jnp.int32),
        tails.astype(jnp.int32),
        entity_table.T,
        jnp.pad(entity_table[FULL_COLS * 128:].T, ((0, 0), (0, 128 - TAIL))).reshape(-1),
        relation_table.T,
    )
